# baseline (speedup 1.0000x reference)
import dataclasses
import functools

import jax
import jax.numpy as jnp
from jax import lax
from jax.experimental import pallas as pl
from jax.experimental.pallas import tpu as pltpu
from jax.experimental.pallas import tpu_sc as plsc

F = 26
B = 4096
S = F * B
NW = 32
SEG_W = S // NW
I_W = B // NW
LANES = 16
NVEC = SEG_W // LANES
MAXK = 7
CAP = SEG_W * MAXK
GROUP = 13
ROWSX = 184
CAPX = ROWSX * 128
VGROUP = 8
VGCH = VGROUP * 128
DUMPW = 1024

_mesh = plsc.VectorSubcoreMesh(core_axis_name="c", subcore_axis_name="s")

_cp = pltpu.CompilerParams()
if "needs_layout_passes" in pltpu.CompilerParams.__dataclass_fields__:
    _cp = dataclasses.replace(_cp, needs_layout_passes=False)


def _wid():
    return lax.axis_index("s") * 2 + lax.axis_index("c")


def _build_sel(perm_v, sel_v):
    @pl.loop(0, NVEC)
    def _(t):
        r = t * LANES + lax.iota(jnp.int32, LANES)
        li = r // F
        f = r - li * F
        p = plsc.load_gather(perm_v, [li])
        sel_v[pl.ds(t * LANES, LANES)] = f * B + p


def _k1_body(lengths_hbm, perm_hbm, lts_hbm, sums_hbm,
             perm_v, sel_v, len_v, lts_v, row_v, sem):
    w = _wid()
    hp = pltpu.async_copy(perm_hbm.at[pl.ds(w * I_W, I_W)], perm_v, sem)
    hl = pltpu.async_copy(lengths_hbm.at[pl.ds(w * SEG_W, SEG_W)], len_v, sem)
    hp.wait()
    hl.wait()
    _build_sel(perm_v, sel_v)
    for g0 in range(0, SEG_W // 128, GROUP):
        hs = [pltpu.async_copy(lengths_hbm.at[sel_v.at[pl.ds(i * 128, 128)]],
                               lts_v.at[pl.ds(i * 128, 128)], sem)
              for i in range(g0, min(g0 + GROUP, SEG_W // 128))]
        for h in hs:
            h.wait()

    def body(t, acc):
        a, b = acc
        return (a + len_v[pl.ds(t * LANES, LANES)],
                b + lts_v[pl.ds(t * LANES, LANES)])

    z = jnp.zeros((LANES,), jnp.int32)
    a, b = lax.fori_loop(0, NVEC, body, (z, z))
    row_v[...] = a
    pltpu.sync_copy(row_v, sums_hbm.at[0, w])
    row_v[...] = b
    pltpu.sync_copy(row_v, sums_hbm.at[1, w])
    pltpu.sync_copy(lts_v, lts_hbm.at[pl.ds(w * SEG_W, SEG_W)])


def _k2_body(lengths_hbm, sums_hbm, instarts_hbm,
             len_v, sums_v, starts_v, sem):
    w = _wid()
    h1 = pltpu.async_copy(sums_hbm.at[0], sums_v, sem)
    h2 = pltpu.async_copy(lengths_hbm.at[pl.ds(w * SEG_W, SEG_W)], len_v, sem)
    h1.wait()
    h2.wait()
    acc = jnp.zeros((LANES,), jnp.int32)
    for r in range(NW):
        acc = acc + sums_v[r] * (w > r).astype(jnp.int32)
    base = jnp.sum(acc)

    def body(t, carry):
        lvec = len_v[pl.ds(t * LANES, LANES)]
        incl = plsc.cumsum(lvec)
        starts_v[pl.ds(t * LANES, LANES)] = carry + incl - lvec
        return carry + jnp.sum(lvec)

    lax.fori_loop(0, NVEC, body, base)
    pltpu.sync_copy(starts_v, instarts_hbm.at[pl.ds(w * SEG_W, SEG_W)])


def _make_k3_body(n_total, clamp):
    def _k3_body(values_hbm, lts_hbm, instarts_hbm, perm_hbm, sums_hbm,
                 outp_hbm, perm_v, sel_v, lts_v, src_v, sums_v,
                 gidx_v, vals_v, fdst_v, fdstt_v, fgi_v, fval_v, sem):
        w = _wid()
        h1 = pltpu.async_copy(perm_hbm.at[pl.ds(w * I_W, I_W)], perm_v, sem)
        h2 = pltpu.async_copy(lts_hbm.at[pl.ds(w * SEG_W, SEG_W)], lts_v, sem)
        h3 = pltpu.async_copy(sums_hbm.at[1], sums_v, sem)
        h1.wait()
        h2.wait()
        h3.wait()
        _build_sel(perm_v, sel_v)
        for g0 in range(0, SEG_W // 128, GROUP):
            hs = [pltpu.async_copy(
                      instarts_hbm.at[sel_v.at[pl.ds(i * 128, 128)]],
                      src_v.at[pl.ds(i * 128, 128)], sem)
                  for i in range(g0, min(g0 + GROUP, SEG_W // 128))]
            for h in hs:
                h.wait()
        accb = jnp.zeros((LANES,), jnp.int32)
        accm = jnp.zeros((LANES,), jnp.int32)
        for r in range(NW):
            row = sums_v[r]
            accb = accb + row * (w > r).astype(jnp.int32)
            accm = accm + row * (w == r).astype(jnp.int32)
        base_out = jnp.sum(accb)
        my_count = jnp.sum(accm)

        pad = base_out % 8
        end = base_out + my_count
        start_al = base_out + ((8 - pad) % 8)
        end_al = end - (end % 8)
        nact = my_count + pad

        n_vec_active = ((nact + VGCH - 1) // VGCH) * (VGCH // LANES)

        def initbody(t, carry):
            posv = t * LANES + lax.iota(jnp.int32, LANES)
            gidx_v[pl.ds(t * LANES, LANES)] = posv % clamp
            return carry

        lax.fori_loop(0, n_vec_active, initbody, 0)

        def expbody(t, carry):
            lvec = lts_v[pl.ds(t * LANES, LANES)]
            svec = src_v[pl.ds(t * LANES, LANES)]
            incl = plsc.cumsum(lvec)
            excl = carry + incl - lvec
            for k in range(MAXK):
                plsc.store_scatter(gidx_v, [excl + k], svec + k,
                                   mask=lvec > k)
            return carry + jnp.sum(lvec)

        lax.fori_loop(0, NVEC, expbody, pad)

        @pl.loop(0, ROWSX, step=VGROUP)
        def _(g):
            @pl.when(g * 128 < nact)
            def _():
                gh = [pltpu.async_copy(
                          values_hbm.at[gidx_v.at[pl.ds((g + kk) * 128, 128)]],
                          vals_v.at[pl.ds((g + kk) * 128, 128)], sem)
                      for kk in range(VGROUP)]
                for h in gh:
                    h.wait()

        i16 = lax.iota(jnp.int32, LANES)
        dump = n_total + w * 32 + i16

        headlen = jnp.minimum((8 - pad) % 8, my_count)
        hvalid = (i16 >= pad) & (i16 < pad + headlen)
        fdst_v[...] = jnp.where(hvalid, base_out + i16 - pad, dump)

        taillen = end % 8
        tvalid = (i16 < taillen) & (end_al >= start_al)
        tpos = jnp.clip(end_al - base_out + pad + i16, 0, CAPX - 1)
        tgi = plsc.load_gather(gidx_v, [tpos])
        fgi_v[...] = jnp.where(tvalid, tgi, 0)
        fdstt_v[...] = jnp.where(tvalid, end_al + i16, dump)

        hh = pltpu.async_copy(vals_v.at[pl.ds(0, LANES)],
                              outp_hbm.at[fdst_v], sem)
        hg = pltpu.async_copy(values_hbm.at[fgi_v], fval_v, sem)

        l8 = jnp.maximum(end_al - start_al, 0)
        ls = jnp.where(pad > 0, 8, 0)

        def chunk_descr(i, chunk, loff, goff):
            return (vals_v.at[pl.ds(pl.multiple_of(loff + i * chunk, 8),
                                    chunk)],
                    outp_hbm.at[pl.ds(pl.multiple_of(goff + i * chunk, 8),
                                      chunk)])

        def fire_chunks(nchunks, chunk, loff, goff):
            def cbody(i, carry):
                pltpu.async_copy(*chunk_descr(i, chunk, loff, goff), sem)
                return carry
            lax.fori_loop(0, nchunks, cbody, 0)

        def drain_chunks(nchunks, chunk, loff, goff):
            def cbody(i, carry):
                pltpu.make_async_copy(*chunk_descr(i, chunk, loff, goff),
                                      sem).wait()
                return carry
            lax.fori_loop(0, nchunks, cbody, 0)

        n_a = l8 // 1024
        off_a = n_a * 1024
        n_b = (l8 - off_a) // 128
        off_b = off_a + n_b * 128
        n_c = (l8 - off_b) // 8
        fire_chunks(n_a, 1024, ls, start_al)
        fire_chunks(n_b, 128, ls + off_a, start_al + off_a)
        fire_chunks(n_c, 8, ls + off_b, start_al + off_b)

        hh.wait()
        hg.wait()
        drain_chunks(n_a, 1024, ls, start_al)
        drain_chunks(n_b, 128, ls + off_a, start_al + off_a)
        drain_chunks(n_c, 8, ls + off_b, start_al + off_b)

        pltpu.async_copy(fval_v, outp_hbm.at[fdstt_v], sem).wait()

    return _k3_body


@functools.partial(
    pl.kernel, mesh=_mesh, compiler_params=_cp,
    out_type=(jax.ShapeDtypeStruct((S,), jnp.int32),
              jax.ShapeDtypeStruct((2, NW, LANES), jnp.int32)),
    scratch_types=[pltpu.VMEM((I_W,), jnp.int32),
                   pltpu.VMEM((SEG_W,), jnp.int32),
                   pltpu.VMEM((SEG_W,), jnp.int32),
                   pltpu.VMEM((SEG_W,), jnp.int32),
                   pltpu.VMEM((LANES,), jnp.int32),
                   pltpu.SemaphoreType.DMA])
def _k1(lengths_hbm, perm_hbm, lts_hbm, sums_hbm, *rest):
    _k1_body(lengths_hbm, perm_hbm, lts_hbm, sums_hbm, *rest)


@functools.partial(
    pl.kernel, mesh=_mesh, compiler_params=_cp,
    out_type=jax.ShapeDtypeStruct((S,), jnp.int32),
    scratch_types=[pltpu.VMEM((SEG_W,), jnp.int32),
                   pltpu.VMEM((NW, LANES), jnp.int32),
                   pltpu.VMEM((SEG_W,), jnp.int32),
                   pltpu.SemaphoreType.DMA])
def _k2(lengths_hbm, sums_hbm, instarts_hbm, *rest):
    _k2_body(lengths_hbm, sums_hbm, instarts_hbm, *rest)


def kernel(values, lengths, permute_idx):
    n_total = values.shape[0]
    lts, sums = _k1(lengths, permute_idx)
    if n_total == 0:
        return jnp.zeros((0,), jnp.float32), lts
    instarts = _k2(lengths, sums)

    clamp = min(DUMPW, n_total)
    k3 = functools.partial(
        pl.kernel, mesh=_mesh, compiler_params=_cp,
        out_type=jax.ShapeDtypeStruct((n_total + DUMPW,), jnp.float32),
        scratch_types=[pltpu.VMEM((I_W,), jnp.int32),
                       pltpu.VMEM((SEG_W,), jnp.int32),
                       pltpu.VMEM((SEG_W,), jnp.int32),
                       pltpu.VMEM((SEG_W,), jnp.int32),
                       pltpu.VMEM((NW, LANES), jnp.int32),
                       pltpu.VMEM((CAPX,), jnp.int32),
                       pltpu.VMEM((CAPX,), jnp.float32),
                       pltpu.VMEM((LANES,), jnp.int32),
                       pltpu.VMEM((LANES,), jnp.int32),
                       pltpu.VMEM((LANES,), jnp.int32),
                       pltpu.VMEM((LANES,), jnp.float32),
                       pltpu.SemaphoreType.DMA])(
        _make_k3_body(n_total, clamp))
    outp = k3(values, lts, instarts, permute_idx, sums)
    return outp[:n_total], lts


# speedup vs baseline: 1.0803x; 1.0803x over previous
import dataclasses
import functools

import jax
import jax.numpy as jnp
from jax import lax
from jax.experimental import pallas as pl
from jax.experimental.pallas import tpu as pltpu
from jax.experimental.pallas import tpu_sc as plsc

F = 26
B = 4096
S = F * B
NW = 32
SEG_W = S // NW
I_W = B // NW
LANES = 16
NVEC = SEG_W // LANES
MAXK = 7
CAP = SEG_W * MAXK
GROUP = 13
ROWSX = 184
CAPX = ROWSX * 128
VGROUP = 23
VGCH = VGROUP * 128
DUMPW = 1024

_mesh = plsc.VectorSubcoreMesh(core_axis_name="c", subcore_axis_name="s")

_cp = pltpu.CompilerParams()
if "needs_layout_passes" in pltpu.CompilerParams.__dataclass_fields__:
    _cp = dataclasses.replace(_cp, needs_layout_passes=False)


def _wid():
    return lax.axis_index("s") * 2 + lax.axis_index("c")


def _build_sel(perm_v, sel_v):
    @pl.loop(0, NVEC)
    def _(t):
        r = t * LANES + lax.iota(jnp.int32, LANES)
        li = r // F
        f = r - li * F
        p = plsc.load_gather(perm_v, [li])
        sel_v[pl.ds(t * LANES, LANES)] = f * B + p


def _k1_body(lengths_hbm, perm_hbm, lts_hbm, sums_hbm,
             perm_v, sel_v, len_v, lts_v, row_v, sem):
    w = _wid()
    hp = pltpu.async_copy(perm_hbm.at[pl.ds(w * I_W, I_W)], perm_v, sem)
    hl = pltpu.async_copy(lengths_hbm.at[pl.ds(w * SEG_W, SEG_W)], len_v, sem)
    hp.wait()
    hl.wait()
    _build_sel(perm_v, sel_v)
    for g0 in range(0, SEG_W // 128, GROUP):
        hs = [pltpu.async_copy(lengths_hbm.at[sel_v.at[pl.ds(i * 128, 128)]],
                               lts_v.at[pl.ds(i * 128, 128)], sem)
              for i in range(g0, min(g0 + GROUP, SEG_W // 128))]
        for h in hs:
            h.wait()

    def body(t, acc):
        a, b = acc
        return (a + len_v[pl.ds(t * LANES, LANES)],
                b + lts_v[pl.ds(t * LANES, LANES)])

    z = jnp.zeros((LANES,), jnp.int32)
    a, b = lax.fori_loop(0, NVEC, body, (z, z))
    row_v[...] = a
    pltpu.sync_copy(row_v, sums_hbm.at[0, w])
    row_v[...] = b
    pltpu.sync_copy(row_v, sums_hbm.at[1, w])
    pltpu.sync_copy(lts_v, lts_hbm.at[pl.ds(w * SEG_W, SEG_W)])


def _k2_body(lengths_hbm, sums_hbm, instarts_hbm,
             len_v, sums_v, starts_v, sem):
    w = _wid()
    h1 = pltpu.async_copy(sums_hbm.at[0], sums_v, sem)
    h2 = pltpu.async_copy(lengths_hbm.at[pl.ds(w * SEG_W, SEG_W)], len_v, sem)
    h1.wait()
    h2.wait()
    acc = jnp.zeros((LANES,), jnp.int32)
    for r in range(NW):
        acc = acc + sums_v[r] * (w > r).astype(jnp.int32)
    base = jnp.sum(acc)

    def body(t, carry):
        lvec = len_v[pl.ds(t * LANES, LANES)]
        incl = plsc.cumsum(lvec)
        starts_v[pl.ds(t * LANES, LANES)] = carry + incl - lvec
        return carry + jnp.sum(lvec)

    lax.fori_loop(0, NVEC, body, base)
    pltpu.sync_copy(starts_v, instarts_hbm.at[pl.ds(w * SEG_W, SEG_W)])


def _make_k3_body(n_total, clamp):
    def _k3_body(values_hbm, lts_hbm, instarts_hbm, perm_hbm, sums_hbm,
                 outp_hbm, perm_v, sel_v, lts_v, src_v, sums_v,
                 gidx_v, vals_v, fdst_v, fdstt_v, fgi_v, fval_v, sem):
        w = _wid()
        h1 = pltpu.async_copy(perm_hbm.at[pl.ds(w * I_W, I_W)], perm_v, sem)
        h2 = pltpu.async_copy(lts_hbm.at[pl.ds(w * SEG_W, SEG_W)], lts_v, sem)
        h3 = pltpu.async_copy(sums_hbm.at[1], sums_v, sem)
        h1.wait()
        h2.wait()
        h3.wait()
        _build_sel(perm_v, sel_v)
        for g0 in range(0, SEG_W // 128, GROUP):
            hs = [pltpu.async_copy(
                      instarts_hbm.at[sel_v.at[pl.ds(i * 128, 128)]],
                      src_v.at[pl.ds(i * 128, 128)], sem)
                  for i in range(g0, min(g0 + GROUP, SEG_W // 128))]
            for h in hs:
                h.wait()
        accb = jnp.zeros((LANES,), jnp.int32)
        accm = jnp.zeros((LANES,), jnp.int32)
        for r in range(NW):
            row = sums_v[r]
            accb = accb + row * (w > r).astype(jnp.int32)
            accm = accm + row * (w == r).astype(jnp.int32)
        base_out = jnp.sum(accb)
        my_count = jnp.sum(accm)

        pad = base_out % 8
        end = base_out + my_count
        start_al = base_out + ((8 - pad) % 8)
        end_al = end - (end % 8)
        nact = my_count + pad

        n_vec_active = ((nact + VGCH - 1) // VGCH) * (VGCH // LANES)

        def initbody(t, carry):
            posv = t * LANES + lax.iota(jnp.int32, LANES)
            gidx_v[pl.ds(t * LANES, LANES)] = posv % clamp
            return carry

        lax.fori_loop(0, n_vec_active, initbody, 0)

        def expbody(t, carry):
            lvec = lts_v[pl.ds(t * LANES, LANES)]
            svec = src_v[pl.ds(t * LANES, LANES)]
            incl = plsc.cumsum(lvec)
            excl = carry + incl - lvec
            for k in range(MAXK):
                plsc.store_scatter(gidx_v, [excl + k], svec + k,
                                   mask=lvec > k)
            return carry + jnp.sum(lvec)

        lax.fori_loop(0, NVEC, expbody, pad)

        @pl.loop(0, ROWSX, step=VGROUP)
        def _(g):
            @pl.when(g * 128 < nact)
            def _():
                gh = [pltpu.async_copy(
                          values_hbm.at[gidx_v.at[pl.ds((g + kk) * 128, 128)]],
                          vals_v.at[pl.ds((g + kk) * 128, 128)], sem)
                      for kk in range(VGROUP)]
                for h in gh:
                    h.wait()

        i16 = lax.iota(jnp.int32, LANES)
        dump = n_total + w * 32 + i16

        headlen = jnp.minimum((8 - pad) % 8, my_count)
        hvalid = (i16 >= pad) & (i16 < pad + headlen)
        fdst_v[...] = jnp.where(hvalid, base_out + i16 - pad, dump)

        taillen = end % 8
        tvalid = (i16 < taillen) & (end_al >= start_al)
        tpos = jnp.clip(end_al - base_out + pad + i16, 0, CAPX - 1)
        tgi = plsc.load_gather(gidx_v, [tpos])
        fgi_v[...] = jnp.where(tvalid, tgi, 0)
        fdstt_v[...] = jnp.where(tvalid, end_al + i16, dump)

        hh = pltpu.async_copy(vals_v.at[pl.ds(0, LANES)],
                              outp_hbm.at[fdst_v], sem)
        hg = pltpu.async_copy(values_hbm.at[fgi_v], fval_v, sem)

        l8 = jnp.maximum(end_al - start_al, 0)
        ls = jnp.where(pad > 0, 8, 0)

        def chunk_descr(i, chunk, loff, goff):
            return (vals_v.at[pl.ds(pl.multiple_of(loff + i * chunk, 8),
                                    chunk)],
                    outp_hbm.at[pl.ds(pl.multiple_of(goff + i * chunk, 8),
                                      chunk)])

        def fire_chunks(nchunks, chunk, loff, goff):
            def cbody(i, carry):
                pltpu.async_copy(*chunk_descr(i, chunk, loff, goff), sem)
                return carry
            lax.fori_loop(0, nchunks, cbody, 0)

        def drain_chunks(nchunks, chunk, loff, goff):
            def cbody(i, carry):
                pltpu.make_async_copy(*chunk_descr(i, chunk, loff, goff),
                                      sem).wait()
                return carry
            lax.fori_loop(0, nchunks, cbody, 0)

        n_a = l8 // 1024
        off_a = n_a * 1024
        n_b = (l8 - off_a) // 128
        off_b = off_a + n_b * 128
        n_c = (l8 - off_b) // 8
        fire_chunks(n_a, 1024, ls, start_al)
        fire_chunks(n_b, 128, ls + off_a, start_al + off_a)
        fire_chunks(n_c, 8, ls + off_b, start_al + off_b)

        hh.wait()
        hg.wait()
        drain_chunks(n_a, 1024, ls, start_al)
        drain_chunks(n_b, 128, ls + off_a, start_al + off_a)
        drain_chunks(n_c, 8, ls + off_b, start_al + off_b)

        pltpu.async_copy(fval_v, outp_hbm.at[fdstt_v], sem).wait()

    return _k3_body


@functools.partial(
    pl.kernel, mesh=_mesh, compiler_params=_cp,
    out_type=(jax.ShapeDtypeStruct((S,), jnp.int32),
              jax.ShapeDtypeStruct((2, NW, LANES), jnp.int32)),
    scratch_types=[pltpu.VMEM((I_W,), jnp.int32),
                   pltpu.VMEM((SEG_W,), jnp.int32),
                   pltpu.VMEM((SEG_W,), jnp.int32),
                   pltpu.VMEM((SEG_W,), jnp.int32),
                   pltpu.VMEM((LANES,), jnp.int32),
                   pltpu.SemaphoreType.DMA])
def _k1(lengths_hbm, perm_hbm, lts_hbm, sums_hbm, *rest):
    _k1_body(lengths_hbm, perm_hbm, lts_hbm, sums_hbm, *rest)


@functools.partial(
    pl.kernel, mesh=_mesh, compiler_params=_cp,
    out_type=jax.ShapeDtypeStruct((S,), jnp.int32),
    scratch_types=[pltpu.VMEM((SEG_W,), jnp.int32),
                   pltpu.VMEM((NW, LANES), jnp.int32),
                   pltpu.VMEM((SEG_W,), jnp.int32),
                   pltpu.SemaphoreType.DMA])
def _k2(lengths_hbm, sums_hbm, instarts_hbm, *rest):
    _k2_body(lengths_hbm, sums_hbm, instarts_hbm, *rest)


def kernel(values, lengths, permute_idx):
    n_total = values.shape[0]
    lts, sums = _k1(lengths, permute_idx)
    if n_total == 0:
        return jnp.zeros((0,), jnp.float32), lts
    instarts = _k2(lengths, sums)

    clamp = min(DUMPW, n_total)
    k3 = functools.partial(
        pl.kernel, mesh=_mesh, compiler_params=_cp,
        out_type=jax.ShapeDtypeStruct((n_total + DUMPW,), jnp.float32),
        scratch_types=[pltpu.VMEM((I_W,), jnp.int32),
                       pltpu.VMEM((SEG_W,), jnp.int32),
                       pltpu.VMEM((SEG_W,), jnp.int32),
                       pltpu.VMEM((SEG_W,), jnp.int32),
                       pltpu.VMEM((NW, LANES), jnp.int32),
                       pltpu.VMEM((CAPX,), jnp.int32),
                       pltpu.VMEM((CAPX,), jnp.float32),
                       pltpu.VMEM((LANES,), jnp.int32),
                       pltpu.VMEM((LANES,), jnp.int32),
                       pltpu.VMEM((LANES,), jnp.int32),
                       pltpu.VMEM((LANES,), jnp.float32),
                       pltpu.SemaphoreType.DMA])(
        _make_k3_body(n_total, clamp))
    outp = k3(values, lts, instarts, permute_idx, sums)
    return outp[:n_total], lts


# speedup vs baseline: 1.2139x; 1.1237x over previous
import dataclasses
import functools

import jax
import jax.numpy as jnp
from jax import lax
from jax.experimental import pallas as pl
from jax.experimental.pallas import tpu as pltpu
from jax.experimental.pallas import tpu_sc as plsc

F = 26
B = 4096
S = F * B
NW = 32
SEG_W = S // NW
I_W = B // NW
LANES = 16
NVEC = SEG_W // LANES
MAXK = 7
CAP = SEG_W * MAXK
GROUP = 13
ROWSX = 184
CAPX = ROWSX * 128
VGROUP = 23
VGCH = VGROUP * 128
DUMPW = 1024

_mesh = plsc.VectorSubcoreMesh(core_axis_name="c", subcore_axis_name="s")

_cp = pltpu.CompilerParams()
if "needs_layout_passes" in pltpu.CompilerParams.__dataclass_fields__:
    _cp = dataclasses.replace(_cp, needs_layout_passes=False)


def _wid():
    return lax.axis_index("s") * 2 + lax.axis_index("c")


def _build_sel(perm_v, sel_v):
    @pl.loop(0, NVEC)
    def _(t):
        r = t * LANES + lax.iota(jnp.int32, LANES)
        li = r // F
        f = r - li * F
        p = plsc.load_gather(perm_v, [li])
        sel_v[pl.ds(t * LANES, LANES)] = f * B + p


def _k1_body(lengths_hbm, perm_hbm, lts_hbm, sums_hbm,
             perm_v, sel_v, len_v, lts_v, row_v, sem):
    w = _wid()
    hp = pltpu.async_copy(perm_hbm.at[pl.ds(w * I_W, I_W)], perm_v, sem)
    hl = pltpu.async_copy(lengths_hbm.at[pl.ds(w * SEG_W, SEG_W)], len_v, sem)
    hp.wait()
    hl.wait()
    _build_sel(perm_v, sel_v)
    for g0 in range(0, SEG_W // 128, GROUP):
        hs = [pltpu.async_copy(lengths_hbm.at[sel_v.at[pl.ds(i * 128, 128)]],
                               lts_v.at[pl.ds(i * 128, 128)], sem)
              for i in range(g0, min(g0 + GROUP, SEG_W // 128))]
        for h in hs:
            h.wait()

    def body(t, acc):
        a, b = acc
        return (a + len_v[pl.ds(t * LANES, LANES)],
                b + lts_v[pl.ds(t * LANES, LANES)])

    z = jnp.zeros((LANES,), jnp.int32)
    a, b = lax.fori_loop(0, NVEC, body, (z, z))
    row_v[...] = a
    pltpu.sync_copy(row_v, sums_hbm.at[0, w])
    row_v[...] = b
    pltpu.sync_copy(row_v, sums_hbm.at[1, w])
    pltpu.sync_copy(lts_v, lts_hbm.at[pl.ds(w * SEG_W, SEG_W)])


def _k2_body(lengths_hbm, sums_hbm, instarts_hbm,
             len_v, sums_v, starts_v, sem):
    w = _wid()
    h1 = pltpu.async_copy(sums_hbm.at[0], sums_v, sem)
    h2 = pltpu.async_copy(lengths_hbm.at[pl.ds(w * SEG_W, SEG_W)], len_v, sem)
    h1.wait()
    h2.wait()
    acc = jnp.zeros((LANES,), jnp.int32)
    for r in range(NW):
        acc = acc + sums_v[r] * (w > r).astype(jnp.int32)
    base = jnp.sum(acc)

    def body(t, carry):
        lvec = len_v[pl.ds(t * LANES, LANES)]
        incl = plsc.cumsum(lvec)
        starts_v[pl.ds(t * LANES, LANES)] = carry + incl - lvec
        return carry + jnp.sum(lvec)

    lax.fori_loop(0, NVEC, body, base)
    pltpu.sync_copy(starts_v, instarts_hbm.at[pl.ds(w * SEG_W, SEG_W)])


def _make_k3_body(n_total, clamp, np_pad):
    slice_w = np_pad // 16

    def _k3_body(values_hbm, lts_hbm, instarts_hbm, perm_hbm, sums_hbm,
                 outp_hbm, perm_v, sel_v, lts_v, src_v, sums_v,
                 gidx_v, vals_v, fdst_v, fdstt_v, fgi_v, fval_v, vsh,
                 sem, sem2):
        w = _wid()
        sid = lax.axis_index("s")
        hv = pltpu.async_copy(
            values_hbm.at[pl.ds(pl.multiple_of(sid * slice_w, 8), slice_w)],
            vsh.at[pl.ds(pl.multiple_of(sid * slice_w, 8), slice_w)], sem2)
        h1 = pltpu.async_copy(perm_hbm.at[pl.ds(w * I_W, I_W)], perm_v, sem)
        h2 = pltpu.async_copy(lts_hbm.at[pl.ds(w * SEG_W, SEG_W)], lts_v, sem)
        h3 = pltpu.async_copy(sums_hbm.at[1], sums_v, sem)
        h1.wait()
        h2.wait()
        h3.wait()
        _build_sel(perm_v, sel_v)
        for g0 in range(0, SEG_W // 128, GROUP):
            hs = [pltpu.async_copy(
                      instarts_hbm.at[sel_v.at[pl.ds(i * 128, 128)]],
                      src_v.at[pl.ds(i * 128, 128)], sem)
                  for i in range(g0, min(g0 + GROUP, SEG_W // 128))]
            for h in hs:
                h.wait()
        accb = jnp.zeros((LANES,), jnp.int32)
        accm = jnp.zeros((LANES,), jnp.int32)
        for r in range(NW):
            row = sums_v[r]
            accb = accb + row * (w > r).astype(jnp.int32)
            accm = accm + row * (w == r).astype(jnp.int32)
        base_out = jnp.sum(accb)
        my_count = jnp.sum(accm)

        pad = base_out % 8
        end = base_out + my_count
        start_al = base_out + ((8 - pad) % 8)
        end_al = end - (end % 8)
        nact = my_count + pad

        n_vec_active = ((nact + VGCH - 1) // VGCH) * (VGCH // LANES)

        def initbody(t, carry):
            posv = t * LANES + lax.iota(jnp.int32, LANES)
            gidx_v[pl.ds(t * LANES, LANES)] = posv % clamp
            return carry

        lax.fori_loop(0, n_vec_active, initbody, 0)

        def expbody(t, carry):
            lvec = lts_v[pl.ds(t * LANES, LANES)]
            svec = src_v[pl.ds(t * LANES, LANES)]
            incl = plsc.cumsum(lvec)
            excl = carry + incl - lvec
            for k in range(MAXK):
                plsc.store_scatter(gidx_v, [excl + k], svec + k,
                                   mask=lvec > k)
            return carry + jnp.sum(lvec)

        lax.fori_loop(0, NVEC, expbody, pad)

        hv.wait()
        plsc.subcore_barrier()

        @pl.loop(0, ROWSX, step=VGROUP)
        def _(g):
            @pl.when(g * 128 < nact)
            def _():
                gh = [pltpu.async_copy(
                          vsh.at[gidx_v.at[pl.ds((g + kk) * 128, 128)]],
                          vals_v.at[pl.ds((g + kk) * 128, 128)], sem)
                      for kk in range(VGROUP)]
                for h in gh:
                    h.wait()

        i16 = lax.iota(jnp.int32, LANES)
        dump = n_total + w * 32 + i16

        headlen = jnp.minimum((8 - pad) % 8, my_count)
        hvalid = (i16 >= pad) & (i16 < pad + headlen)
        fdst_v[...] = jnp.where(hvalid, base_out + i16 - pad, dump)

        taillen = end % 8
        tvalid = (i16 < taillen) & (end_al >= start_al)
        tpos = jnp.clip(end_al - base_out + pad + i16, 0, CAPX - 1)
        tgi = plsc.load_gather(gidx_v, [tpos])
        fgi_v[...] = jnp.where(tvalid, tgi, 0)
        fdstt_v[...] = jnp.where(tvalid, end_al + i16, dump)

        hh = pltpu.async_copy(vals_v.at[pl.ds(0, LANES)],
                              outp_hbm.at[fdst_v], sem)
        hg = pltpu.async_copy(values_hbm.at[fgi_v], fval_v, sem)

        l8 = jnp.maximum(end_al - start_al, 0)
        ls = jnp.where(pad > 0, 8, 0)

        def chunk_descr(i, chunk, loff, goff):
            return (vals_v.at[pl.ds(pl.multiple_of(loff + i * chunk, 8),
                                    chunk)],
                    outp_hbm.at[pl.ds(pl.multiple_of(goff + i * chunk, 8),
                                      chunk)])

        def fire_chunks(nchunks, chunk, loff, goff):
            def cbody(i, carry):
                pltpu.async_copy(*chunk_descr(i, chunk, loff, goff), sem)
                return carry
            lax.fori_loop(0, nchunks, cbody, 0)

        def drain_chunks(nchunks, chunk, loff, goff):
            def cbody(i, carry):
                pltpu.make_async_copy(*chunk_descr(i, chunk, loff, goff),
                                      sem).wait()
                return carry
            lax.fori_loop(0, nchunks, cbody, 0)

        n_a = l8 // 1024
        off_a = n_a * 1024
        n_b = (l8 - off_a) // 128
        off_b = off_a + n_b * 128
        n_c = (l8 - off_b) // 8
        fire_chunks(n_a, 1024, ls, start_al)
        fire_chunks(n_b, 128, ls + off_a, start_al + off_a)
        fire_chunks(n_c, 8, ls + off_b, start_al + off_b)

        hh.wait()
        hg.wait()
        drain_chunks(n_a, 1024, ls, start_al)
        drain_chunks(n_b, 128, ls + off_a, start_al + off_a)
        drain_chunks(n_c, 8, ls + off_b, start_al + off_b)

        pltpu.async_copy(fval_v, outp_hbm.at[fdstt_v], sem).wait()

    return _k3_body


@functools.partial(
    pl.kernel, mesh=_mesh, compiler_params=_cp,
    out_type=(jax.ShapeDtypeStruct((S,), jnp.int32),
              jax.ShapeDtypeStruct((2, NW, LANES), jnp.int32)),
    scratch_types=[pltpu.VMEM((I_W,), jnp.int32),
                   pltpu.VMEM((SEG_W,), jnp.int32),
                   pltpu.VMEM((SEG_W,), jnp.int32),
                   pltpu.VMEM((SEG_W,), jnp.int32),
                   pltpu.VMEM((LANES,), jnp.int32),
                   pltpu.SemaphoreType.DMA])
def _k1(lengths_hbm, perm_hbm, lts_hbm, sums_hbm, *rest):
    _k1_body(lengths_hbm, perm_hbm, lts_hbm, sums_hbm, *rest)


@functools.partial(
    pl.kernel, mesh=_mesh, compiler_params=_cp,
    out_type=jax.ShapeDtypeStruct((S,), jnp.int32),
    scratch_types=[pltpu.VMEM((SEG_W,), jnp.int32),
                   pltpu.VMEM((NW, LANES), jnp.int32),
                   pltpu.VMEM((SEG_W,), jnp.int32),
                   pltpu.SemaphoreType.DMA])
def _k2(lengths_hbm, sums_hbm, instarts_hbm, *rest):
    _k2_body(lengths_hbm, sums_hbm, instarts_hbm, *rest)


def kernel(values, lengths, permute_idx):
    n_total = values.shape[0]
    lts, sums = _k1(lengths, permute_idx)
    if n_total == 0:
        return jnp.zeros((0,), jnp.float32), lts
    instarts = _k2(lengths, sums)

    clamp = min(DUMPW, n_total)
    np_pad = -(-n_total // 2048) * 2048
    values_p = jnp.concatenate(
        [values, jnp.zeros((np_pad - n_total,), jnp.float32)])
    k3 = functools.partial(
        pl.kernel, mesh=_mesh, compiler_params=_cp,
        out_type=jax.ShapeDtypeStruct((n_total + DUMPW,), jnp.float32),
        scratch_types=[pltpu.VMEM((I_W,), jnp.int32),
                       pltpu.VMEM((SEG_W,), jnp.int32),
                       pltpu.VMEM((SEG_W,), jnp.int32),
                       pltpu.VMEM((SEG_W,), jnp.int32),
                       pltpu.VMEM((NW, LANES), jnp.int32),
                       pltpu.VMEM((CAPX,), jnp.int32),
                       pltpu.VMEM((CAPX,), jnp.float32),
                       pltpu.VMEM((LANES,), jnp.int32),
                       pltpu.VMEM((LANES,), jnp.int32),
                       pltpu.VMEM((LANES,), jnp.int32),
                       pltpu.VMEM((LANES,), jnp.float32),
                       pltpu.VMEM_SHARED((np_pad,), jnp.float32),
                       pltpu.SemaphoreType.DMA,
                       pltpu.SemaphoreType.DMA])(
        _make_k3_body(n_total, clamp, np_pad))
    outp = k3(values_p, lts, instarts, permute_idx, sums)
    return outp[:n_total], lts


# speedup vs baseline: 1.3103x; 1.0794x over previous
import dataclasses
import functools

import jax
import jax.numpy as jnp
from jax import lax
from jax.experimental import pallas as pl
from jax.experimental.pallas import tpu as pltpu
from jax.experimental.pallas import tpu_sc as plsc

F = 26
B = 4096
S = F * B
NW = 32
SEG_W = S // NW
I_W = B // NW
LANES = 16
NVEC = SEG_W // LANES
MAXK = 7
CAP = SEG_W * MAXK
GROUP = 13
ROWSX = 184
CAPX = ROWSX * 128
VGROUP = 23
VGCH = VGROUP * 128
DUMPW = 1024

_mesh = plsc.VectorSubcoreMesh(core_axis_name="c", subcore_axis_name="s")

_cp = pltpu.CompilerParams()
if "needs_layout_passes" in pltpu.CompilerParams.__dataclass_fields__:
    _cp = dataclasses.replace(_cp, needs_layout_passes=False)


def _wid():
    return lax.axis_index("s") * 2 + lax.axis_index("c")


def _build_sel(perm_v, sel_v):
    @plsc.parallel_loop(0, NVEC, unroll=4)
    def _(t):
        r = t * LANES + lax.iota(jnp.int32, LANES)
        li = r // F
        f = r - li * F
        p = plsc.load_gather(perm_v, [li])
        sel_v[pl.ds(t * LANES, LANES)] = f * B + p


def _k1_body(lengths_hbm, perm_hbm, lts_hbm, sums_hbm,
             perm_v, sel_v, len_v, lts_v, row_v, sem):
    w = _wid()
    hp = pltpu.async_copy(perm_hbm.at[pl.ds(w * I_W, I_W)], perm_v, sem)
    hl = pltpu.async_copy(lengths_hbm.at[pl.ds(w * SEG_W, SEG_W)], len_v, sem)
    hp.wait()
    hl.wait()
    _build_sel(perm_v, sel_v)
    for g0 in range(0, SEG_W // 128, GROUP):
        hs = [pltpu.async_copy(lengths_hbm.at[sel_v.at[pl.ds(i * 128, 128)]],
                               lts_v.at[pl.ds(i * 128, 128)], sem)
              for i in range(g0, min(g0 + GROUP, SEG_W // 128))]
        for h in hs:
            h.wait()

    z = jnp.zeros((LANES,), jnp.int32)

    @plsc.parallel_loop(0, NVEC, unroll=4, carry=z)
    def acc_in(t, acc):
        return acc + len_v[pl.ds(t * LANES, LANES)]

    @plsc.parallel_loop(0, NVEC, unroll=4, carry=z)
    def acc_out(t, acc):
        return acc + lts_v[pl.ds(t * LANES, LANES)]

    row_v[...] = acc_in
    pltpu.sync_copy(row_v, sums_hbm.at[0, w])
    row_v[...] = acc_out
    pltpu.sync_copy(row_v, sums_hbm.at[1, w])
    pltpu.sync_copy(lts_v, lts_hbm.at[pl.ds(w * SEG_W, SEG_W)])


def _k2_body(lengths_hbm, sums_hbm, instarts_hbm,
             len_v, sums_v, starts_v, sem):
    w = _wid()
    h1 = pltpu.async_copy(sums_hbm.at[0], sums_v, sem)
    h2 = pltpu.async_copy(lengths_hbm.at[pl.ds(w * SEG_W, SEG_W)], len_v, sem)
    h1.wait()
    h2.wait()
    acc = jnp.zeros((LANES,), jnp.int32)
    for r in range(NW):
        acc = acc + sums_v[r] * (w > r).astype(jnp.int32)
    base = jnp.sum(acc)

    @plsc.parallel_loop(0, NVEC, unroll=2, carry=base)
    def _(t, carry):
        lvec = len_v[pl.ds(t * LANES, LANES)]
        incl = plsc.cumsum(lvec)
        starts_v[pl.ds(t * LANES, LANES)] = carry + incl - lvec
        return carry + jnp.sum(lvec)
    pltpu.sync_copy(starts_v, instarts_hbm.at[pl.ds(w * SEG_W, SEG_W)])


def _make_k3_body(n_total, clamp, np_pad):
    slice_w = np_pad // 16

    def _k3_body(values_hbm, lts_hbm, instarts_hbm, perm_hbm, sums_hbm,
                 outp_hbm, perm_v, sel_v, lts_v, src_v, sums_v,
                 gidx_v, vals_v, fdst_v, fdstt_v, fgi_v, fval_v, vsh,
                 sem, sem2):
        w = _wid()
        sid = lax.axis_index("s")
        hv = pltpu.async_copy(
            values_hbm.at[pl.ds(pl.multiple_of(sid * slice_w, 8), slice_w)],
            vsh.at[pl.ds(pl.multiple_of(sid * slice_w, 8), slice_w)], sem2)
        h1 = pltpu.async_copy(perm_hbm.at[pl.ds(w * I_W, I_W)], perm_v, sem)
        h2 = pltpu.async_copy(lts_hbm.at[pl.ds(w * SEG_W, SEG_W)], lts_v, sem)
        h3 = pltpu.async_copy(sums_hbm.at[1], sums_v, sem)
        h1.wait()
        h2.wait()
        h3.wait()
        _build_sel(perm_v, sel_v)
        for g0 in range(0, SEG_W // 128, GROUP):
            hs = [pltpu.async_copy(
                      instarts_hbm.at[sel_v.at[pl.ds(i * 128, 128)]],
                      src_v.at[pl.ds(i * 128, 128)], sem)
                  for i in range(g0, min(g0 + GROUP, SEG_W // 128))]
            for h in hs:
                h.wait()
        accb = jnp.zeros((LANES,), jnp.int32)
        accm = jnp.zeros((LANES,), jnp.int32)
        for r in range(NW):
            row = sums_v[r]
            accb = accb + row * (w > r).astype(jnp.int32)
            accm = accm + row * (w == r).astype(jnp.int32)
        base_out = jnp.sum(accb)
        my_count = jnp.sum(accm)

        pad = base_out % 8
        end = base_out + my_count
        start_al = base_out + ((8 - pad) % 8)
        end_al = end - (end % 8)
        nact = my_count + pad

        n_vec_active = ((nact + VGCH - 1) // VGCH) * (VGCH // LANES)

        @plsc.parallel_loop(0, n_vec_active, unroll=2)
        def _(t):
            posv = t * LANES + lax.iota(jnp.int32, LANES)
            gidx_v[pl.ds(t * LANES, LANES)] = posv % clamp

        @plsc.parallel_loop(0, NVEC, unroll=2, carry=pad)
        def _(t, carry):
            lvec = lts_v[pl.ds(t * LANES, LANES)]
            svec = src_v[pl.ds(t * LANES, LANES)]
            incl = plsc.cumsum(lvec)
            excl = carry + incl - lvec
            for k in range(MAXK):
                plsc.store_scatter(gidx_v, [excl + k], svec + k,
                                   mask=lvec > k)
            return carry + jnp.sum(lvec)

        hv.wait()
        plsc.subcore_barrier()

        @pl.loop(0, ROWSX, step=VGROUP)
        def _(g):
            @pl.when(g * 128 < nact)
            def _():
                gh = [pltpu.async_copy(
                          vsh.at[gidx_v.at[pl.ds((g + kk) * 128, 128)]],
                          vals_v.at[pl.ds((g + kk) * 128, 128)], sem)
                      for kk in range(VGROUP)]
                for h in gh:
                    h.wait()

        i16 = lax.iota(jnp.int32, LANES)
        dump = n_total + w * 32 + i16

        headlen = jnp.minimum((8 - pad) % 8, my_count)
        hvalid = (i16 >= pad) & (i16 < pad + headlen)
        fdst_v[...] = jnp.where(hvalid, base_out + i16 - pad, dump)

        taillen = end % 8
        tvalid = (i16 < taillen) & (end_al >= start_al)
        tpos = jnp.clip(end_al - base_out + pad + i16, 0, CAPX - 1)
        tgi = plsc.load_gather(gidx_v, [tpos])
        fgi_v[...] = jnp.where(tvalid, tgi, 0)
        fdstt_v[...] = jnp.where(tvalid, end_al + i16, dump)

        hh = pltpu.async_copy(vals_v.at[pl.ds(0, LANES)],
                              outp_hbm.at[fdst_v], sem)
        hg = pltpu.async_copy(values_hbm.at[fgi_v], fval_v, sem)

        l8 = jnp.maximum(end_al - start_al, 0)
        ls = jnp.where(pad > 0, 8, 0)

        def chunk_descr(i, chunk, loff, goff):
            return (vals_v.at[pl.ds(pl.multiple_of(loff + i * chunk, 8),
                                    chunk)],
                    outp_hbm.at[pl.ds(pl.multiple_of(goff + i * chunk, 8),
                                      chunk)])

        def fire_chunks(nchunks, chunk, loff, goff):
            def cbody(i, carry):
                pltpu.async_copy(*chunk_descr(i, chunk, loff, goff), sem)
                return carry
            lax.fori_loop(0, nchunks, cbody, 0)

        def drain_chunks(nchunks, chunk, loff, goff):
            def cbody(i, carry):
                pltpu.make_async_copy(*chunk_descr(i, chunk, loff, goff),
                                      sem).wait()
                return carry
            lax.fori_loop(0, nchunks, cbody, 0)

        n_a = l8 // 1024
        off_a = n_a * 1024
        n_b = (l8 - off_a) // 128
        off_b = off_a + n_b * 128
        n_c = (l8 - off_b) // 8
        fire_chunks(n_a, 1024, ls, start_al)
        fire_chunks(n_b, 128, ls + off_a, start_al + off_a)
        fire_chunks(n_c, 8, ls + off_b, start_al + off_b)

        hh.wait()
        hg.wait()
        drain_chunks(n_a, 1024, ls, start_al)
        drain_chunks(n_b, 128, ls + off_a, start_al + off_a)
        drain_chunks(n_c, 8, ls + off_b, start_al + off_b)

        pltpu.async_copy(fval_v, outp_hbm.at[fdstt_v], sem).wait()

    return _k3_body


@functools.partial(
    pl.kernel, mesh=_mesh, compiler_params=_cp,
    out_type=(jax.ShapeDtypeStruct((S,), jnp.int32),
              jax.ShapeDtypeStruct((2, NW, LANES), jnp.int32)),
    scratch_types=[pltpu.VMEM((I_W,), jnp.int32),
                   pltpu.VMEM((SEG_W,), jnp.int32),
                   pltpu.VMEM((SEG_W,), jnp.int32),
                   pltpu.VMEM((SEG_W,), jnp.int32),
                   pltpu.VMEM((LANES,), jnp.int32),
                   pltpu.SemaphoreType.DMA])
def _k1(lengths_hbm, perm_hbm, lts_hbm, sums_hbm, *rest):
    _k1_body(lengths_hbm, perm_hbm, lts_hbm, sums_hbm, *rest)


@functools.partial(
    pl.kernel, mesh=_mesh, compiler_params=_cp,
    out_type=jax.ShapeDtypeStruct((S,), jnp.int32),
    scratch_types=[pltpu.VMEM((SEG_W,), jnp.int32),
                   pltpu.VMEM((NW, LANES), jnp.int32),
                   pltpu.VMEM((SEG_W,), jnp.int32),
                   pltpu.SemaphoreType.DMA])
def _k2(lengths_hbm, sums_hbm, instarts_hbm, *rest):
    _k2_body(lengths_hbm, sums_hbm, instarts_hbm, *rest)


def kernel(values, lengths, permute_idx):
    n_total = values.shape[0]
    lts, sums = _k1(lengths, permute_idx)
    if n_total == 0:
        return jnp.zeros((0,), jnp.float32), lts
    instarts = _k2(lengths, sums)

    clamp = min(DUMPW, n_total)
    np_pad = -(-n_total // 2048) * 2048
    values_p = jnp.concatenate(
        [values, jnp.zeros((np_pad - n_total,), jnp.float32)])
    k3 = functools.partial(
        pl.kernel, mesh=_mesh, compiler_params=_cp,
        out_type=jax.ShapeDtypeStruct((n_total + DUMPW,), jnp.float32),
        scratch_types=[pltpu.VMEM((I_W,), jnp.int32),
                       pltpu.VMEM((SEG_W,), jnp.int32),
                       pltpu.VMEM((SEG_W,), jnp.int32),
                       pltpu.VMEM((SEG_W,), jnp.int32),
                       pltpu.VMEM((NW, LANES), jnp.int32),
                       pltpu.VMEM((CAPX,), jnp.int32),
                       pltpu.VMEM((CAPX,), jnp.float32),
                       pltpu.VMEM((LANES,), jnp.int32),
                       pltpu.VMEM((LANES,), jnp.int32),
                       pltpu.VMEM((LANES,), jnp.int32),
                       pltpu.VMEM((LANES,), jnp.float32),
                       pltpu.VMEM_SHARED((np_pad,), jnp.float32),
                       pltpu.SemaphoreType.DMA,
                       pltpu.SemaphoreType.DMA])(
        _make_k3_body(n_total, clamp, np_pad))
    outp = k3(values_p, lts, instarts, permute_idx, sums)
    return outp[:n_total], lts
